# Initial kernel scaffold; baseline (speedup 1.0000x reference)
import jax
import jax.numpy as jnp
from jax.experimental import pallas as pl


def kernel(x, edge_index, W1, b1, W2, b2):
    raise NotImplementedError("write your pallas kernel here")


# speedup vs baseline: 1.1657x; 1.1657x over previous
import jax
import jax.numpy as jnp
from jax.experimental import pallas as pl
from jax.experimental.pallas import tpu as pltpu


def _mm_body(x_ref, w_ref, o_ref):
    o_ref[...] = jnp.dot(x_ref[...], w_ref[...], preferred_element_type=jnp.float32)


def _matmul(x, w):
    return pl.pallas_call(
        _mm_body,
        out_shape=jax.ShapeDtypeStruct((x.shape[0], w.shape[1]), jnp.float32),
    )(x, w)


def kernel(x, edge_index, W1, b1, W2, b2):
    N = x.shape[0]
    self_loops = jnp.arange(N, dtype=edge_index.dtype)
    src = jnp.concatenate([edge_index[0], self_loops])
    dst = jnp.concatenate([edge_index[1], self_loops])
    deg = jnp.zeros((N,), dtype=x.dtype).at[dst].add(1.0)
    dinv = jax.lax.rsqrt(jnp.maximum(deg, 1e-12))
    norm = dinv[src] * dinv[dst]

    h = _matmul(x, W1)
    msg = jnp.take(h, src, axis=0) * norm[:, None]
    out1 = jnp.zeros((N, W1.shape[1]), dtype=x.dtype).at[dst].add(msg) + b1
    r = jax.nn.relu(out1)

    h2 = _matmul(r, W2)
    msg2 = jnp.take(h2, src, axis=0) * norm[:, None]
    out2 = jnp.zeros((N, 1), dtype=x.dtype).at[dst].add(msg2) + b2
    return jnp.squeeze(out2, axis=-1)


# speedup vs baseline: 18.9210x; 16.2318x over previous
import functools

import jax
import jax.numpy as jnp
from jax import lax
from jax.experimental import pallas as pl
from jax.experimental.pallas import tpu as pltpu
from jax.experimental.pallas import tpu_sc as plsc

N = 10000
NPAD = 10240
E = 320000
H = 32
NC = 2
NS = 16
CH = 80
EPC = E // NC
EPT = EPC // NS
NCHUNK = EPT // CH
SL = NPAD // NS

_f32 = jnp.float32
_mesh = plsc.VectorSubcoreMesh(core_axis_name="c", subcore_axis_name="s")


def _deg_body(dst_e, degp, dstbuf, ones, zbuf, deg_sh):
    c = lax.axis_index("c")
    s = lax.axis_index("s")

    def _init(i, _):
        ones[pl.ds(i * 16, 16)] = jnp.ones((16,), _f32)
        return 0

    lax.fori_loop(0, CH // 16, _init, 0)

    def _zinit(i, _):
        zbuf[pl.ds(i * 16, 16)] = jnp.zeros((16,), _f32)
        return 0

    lax.fori_loop(0, SL // 16, _zinit, 0)
    pltpu.sync_copy(zbuf, deg_sh.at[pl.ds(s * SL, SL)])
    plsc.subcore_barrier()

    ebase = c * EPC + s * EPT

    def _chunk(i, _):
        pltpu.sync_copy(dst_e.at[pl.ds(ebase + i * CH, CH)], dstbuf)
        pltpu.sync_copy(ones, deg_sh.at[dstbuf], add=True)
        return 0

    lax.fori_loop(0, NCHUNK, _chunk, 0)
    plsc.subcore_barrier()
    pltpu.sync_copy(deg_sh.at[pl.ds(s * SL, SL)], degp.at[c, pl.ds(s * SL, SL)])


_deg_kernel = functools.partial(
    pl.kernel,
    out_type=jax.ShapeDtypeStruct((NC, NPAD), _f32),
    mesh=_mesh,
    scratch_types=[
        pltpu.VMEM((CH,), jnp.int32),
        pltpu.VMEM((CH,), _f32),
        pltpu.VMEM((SL,), _f32),
        pltpu.VMEM_SHARED((NPAD,), _f32),
    ],
)(_deg_body)


def _agg1_body(src_e, dst_e, hs1, zeros2d, aggp, srcbuf, dstbuf, rows, agg_sh):
    c = lax.axis_index("c")
    s = lax.axis_index("s")

    @pl.when(s == 0)
    def _():
        pltpu.sync_copy(zeros2d, agg_sh)

    plsc.subcore_barrier()

    ebase = c * EPC + s * EPT

    def _chunk(i, _):
        pltpu.sync_copy(src_e.at[pl.ds(ebase + i * CH, CH)], srcbuf)
        pltpu.sync_copy(dst_e.at[pl.ds(ebase + i * CH, CH)], dstbuf)
        pltpu.sync_copy(hs1.at[srcbuf], rows)
        pltpu.sync_copy(rows, agg_sh.at[dstbuf], add=True)
        return 0

    lax.fori_loop(0, NCHUNK, _chunk, 0)
    plsc.subcore_barrier()
    pltpu.sync_copy(agg_sh.at[pl.ds(s * SL, SL)], aggp.at[c, pl.ds(s * SL, SL)])


_agg1_kernel = functools.partial(
    pl.kernel,
    out_type=jax.ShapeDtypeStruct((NC, NPAD, H), _f32),
    mesh=_mesh,
    compiler_params=pltpu.CompilerParams(use_tc_tiling_on_sc=False),
    scratch_types=[
        pltpu.VMEM((CH,), jnp.int32),
        pltpu.VMEM((CH,), jnp.int32),
        pltpu.VMEM((CH, H), _f32),
        pltpu.VMEM_SHARED((NPAD, H), _f32),
    ],
)(_agg1_body)


def _agg2_body(src_e, dst_e, hs2, aggp, srcbuf, dstbuf, vals, zbuf, agg_sh):
    c = lax.axis_index("c")
    s = lax.axis_index("s")

    def _zinit(i, _):
        zbuf[pl.ds(i * 16, 16)] = jnp.zeros((16,), _f32)
        return 0

    lax.fori_loop(0, SL // 16, _zinit, 0)
    pltpu.sync_copy(zbuf, agg_sh.at[pl.ds(s * SL, SL)])
    plsc.subcore_barrier()

    ebase = c * EPC + s * EPT

    def _chunk(i, _):
        pltpu.sync_copy(src_e.at[pl.ds(ebase + i * CH, CH)], srcbuf)
        pltpu.sync_copy(dst_e.at[pl.ds(ebase + i * CH, CH)], dstbuf)
        pltpu.sync_copy(hs2.at[srcbuf], vals)
        pltpu.sync_copy(vals, agg_sh.at[dstbuf], add=True)
        return 0

    lax.fori_loop(0, NCHUNK, _chunk, 0)
    plsc.subcore_barrier()
    pltpu.sync_copy(agg_sh.at[pl.ds(s * SL, SL)], aggp.at[c, pl.ds(s * SL, SL)])


_agg2_kernel = functools.partial(
    pl.kernel,
    out_type=jax.ShapeDtypeStruct((NC, NPAD), _f32),
    mesh=_mesh,
    scratch_types=[
        pltpu.VMEM((CH,), jnp.int32),
        pltpu.VMEM((CH,), jnp.int32),
        pltpu.VMEM((CH,), _f32),
        pltpu.VMEM((SL,), _f32),
        pltpu.VMEM_SHARED((NPAD,), _f32),
    ],
)(_agg2_body)


def _b_body(degp_ref, x_ref, w1_ref, hs1_ref, dinv_ref):
    degp = degp_ref[...]
    deg = degp[0] + degp[1] + 1.0
    dinv = lax.rsqrt(jnp.maximum(deg, 1e-12))
    dinv_ref[...] = dinv
    h1 = jnp.dot(x_ref[...], w1_ref[...], preferred_element_type=_f32)
    hs1_ref[...] = h1 * dinv[:N][:, None]


def _d_body(aggp_ref, hs1_ref, dinv_ref, w2_ref, b1_ref, b2_ref, hs2_ref, c2_ref):
    hs1 = hs1_ref[...]
    dinv = dinv_ref[...][:N]
    a = aggp_ref[0, :N] + aggp_ref[1, :N] + hs1
    out1 = dinv[:, None] * a + b1_ref[...]
    r = jnp.maximum(out1, 0.0)
    h2 = jnp.dot(r, w2_ref[...], preferred_element_type=_f32)[:, 0]
    hs2 = dinv * h2
    hs2_ref[...] = hs2
    c2_ref[...] = dinv * hs2 + b2_ref[...]


def _f_body(aggp_ref, dinv_ref, c2_ref, out_ref):
    dinv = dinv_ref[...][:N]
    out_ref[...] = dinv * (aggp_ref[0, :N] + aggp_ref[1, :N]) + c2_ref[...]


def kernel(x, edge_index, W1, b1, W2, b2):
    src_e = edge_index[0]
    dst_e = edge_index[1]
    degp = _deg_kernel(dst_e)

    hs1, dinv = pl.pallas_call(
        _b_body,
        out_shape=(
            jax.ShapeDtypeStruct((N, H), _f32),
            jax.ShapeDtypeStruct((NPAD,), _f32),
        ),
    )(degp, x, W1)

    zeros2d = jnp.zeros((NPAD, H), _f32)
    agg1p = _agg1_kernel(src_e, dst_e, hs1, zeros2d)

    hs2, c2 = pl.pallas_call(
        _d_body,
        out_shape=(
            jax.ShapeDtypeStruct((N,), _f32),
            jax.ShapeDtypeStruct((N,), _f32),
        ),
    )(agg1p, hs1, dinv, W2, b1, b2)

    agg2p = _agg2_kernel(src_e, dst_e, hs2)

    out = pl.pallas_call(
        _f_body,
        out_shape=jax.ShapeDtypeStruct((N,), _f32),
    )(agg2p, dinv, c2)
    return out
